# baseline (device time: 114411 ns/iter reference)
import jax
import jax.numpy as jnp
from jax import lax
from jax.experimental import pallas as pl
from jax.experimental.pallas import tpu as pltpu

SIZES = [256] * 7 + [128, 64, 32, 16, 16]
C = len(SIZES)
OFFS = [sum(SIZES[:i]) for i in range(C)]


def kernel(x):
    m, n = x.shape
    half = m // 2
    assert sum(SIZES) == half

    def body(x_ref, out_ref, xh_ref, comm_ref, red_ref,
             li, lo, s1, r1, s2, r2):
        my_x = lax.axis_index("x")
        my_y = lax.axis_index("y")
        x_peer = (1 - my_x, my_y)
        y_peer = (my_x, 1 - my_y)

        barrier_sem = pltpu.get_barrier_semaphore()
        for nbr in (x_peer, y_peer):
            pl.semaphore_signal(
                barrier_sem, inc=1,
                device_id=nbr, device_id_type=pl.DeviceIdType.MESH,
            )
        pl.semaphore_wait(barrier_sem, 2)

        mine0 = my_y * half
        other0 = (1 - my_y) * half

        p1 = []
        lcp = []
        for c in range(C):
            rows = pl.ds(mine0 + OFFS[c], SIZES[c])
            crows = pl.ds(OFFS[c], SIZES[c])
            rdma = pltpu.make_async_remote_copy(
                src_ref=x_ref.at[rows, :],
                dst_ref=comm_ref.at[crows, :],
                send_sem=s1.at[c],
                recv_sem=r1.at[c],
                device_id=x_peer,
                device_id_type=pl.DeviceIdType.MESH,
            )
            rdma.start()
            p1.append(rdma)
            cp = pltpu.make_async_copy(
                x_ref.at[rows, :], xh_ref.at[crows, :], li.at[c]
            )
            cp.start()
            lcp.append(cp)

        p2 = []
        ocp = []
        for c in range(C):
            p1[c].wait_recv()
            lcp[c].wait()
            sl = (pl.ds(OFFS[c], SIZES[c]), slice(None))
            red_ref[sl] = xh_ref[sl] + comm_ref[sl]
            rows = pl.ds(mine0 + OFFS[c], SIZES[c])
            crows = pl.ds(OFFS[c], SIZES[c])
            rdma = pltpu.make_async_remote_copy(
                src_ref=red_ref.at[crows, :],
                dst_ref=out_ref.at[rows, :],
                send_sem=s2.at[c],
                recv_sem=r2.at[c],
                device_id=y_peer,
                device_id_type=pl.DeviceIdType.MESH,
            )
            rdma.start()
            p2.append(rdma)
            cp = pltpu.make_async_copy(
                red_ref.at[crows, :], out_ref.at[rows, :], lo.at[c]
            )
            cp.start()
            ocp.append(cp)

        for c in range(C):
            rows = pl.ds(other0 + OFFS[c], SIZES[c])
            recv = pltpu.make_async_remote_copy(
                src_ref=red_ref.at[pl.ds(OFFS[c], SIZES[c]), :],
                dst_ref=out_ref.at[rows, :],
                send_sem=s2.at[c],
                recv_sem=r2.at[c],
                device_id=y_peer,
                device_id_type=pl.DeviceIdType.MESH,
            )
            recv.wait_recv()

        for c in range(C):
            ocp[c].wait()
            p1[c].wait_send()
            p2[c].wait_send()

    return pl.pallas_call(
        body,
        out_shape=jax.ShapeDtypeStruct((m, n), x.dtype),
        in_specs=[pl.BlockSpec(memory_space=pl.ANY)],
        out_specs=pl.BlockSpec(memory_space=pl.ANY),
        scratch_shapes=[
            pltpu.VMEM((half, n), x.dtype),
            pltpu.VMEM((half, n), x.dtype),
            pltpu.VMEM((half, n), x.dtype),
            pltpu.SemaphoreType.DMA((C,)),
            pltpu.SemaphoreType.DMA((C,)),
            pltpu.SemaphoreType.DMA((C,)),
            pltpu.SemaphoreType.DMA((C,)),
            pltpu.SemaphoreType.DMA((C,)),
            pltpu.SemaphoreType.DMA((C,)),
        ],
        compiler_params=pltpu.CompilerParams(collective_id=0),
    )(x)


# device time: 107077 ns/iter; 1.0685x vs baseline; 1.0685x over previous
import jax
import jax.numpy as jnp
from jax import lax
from jax.experimental import pallas as pl
from jax.experimental.pallas import tpu as pltpu

SIZES = [64] * 31 + [32, 16, 16]
C = len(SIZES)
OFFS = [sum(SIZES[:i]) for i in range(C)]


def kernel(x):
    m, n = x.shape
    half = m // 2
    assert sum(SIZES) == half

    def body(x_ref, out_ref, xh_ref, comm_ref, red_ref,
             li, lo, s1, r1, s2, r2):
        my_x = lax.axis_index("x")
        my_y = lax.axis_index("y")
        x_peer = (1 - my_x, my_y)
        y_peer = (my_x, 1 - my_y)

        barrier_sem = pltpu.get_barrier_semaphore()
        for nbr in (x_peer, y_peer):
            pl.semaphore_signal(
                barrier_sem, inc=1,
                device_id=nbr, device_id_type=pl.DeviceIdType.MESH,
            )
        pl.semaphore_wait(barrier_sem, 2)

        mine0 = my_y * half
        other0 = (1 - my_y) * half

        p1 = []
        lcp = []
        for c in range(C):
            rows = pl.ds(mine0 + OFFS[c], SIZES[c])
            crows = pl.ds(OFFS[c], SIZES[c])
            rdma = pltpu.make_async_remote_copy(
                src_ref=x_ref.at[rows, :],
                dst_ref=comm_ref.at[crows, :],
                send_sem=s1.at[c],
                recv_sem=r1.at[c],
                device_id=x_peer,
                device_id_type=pl.DeviceIdType.MESH,
            )
            rdma.start()
            p1.append(rdma)
            cp = pltpu.make_async_copy(
                x_ref.at[rows, :], xh_ref.at[crows, :], li.at[c]
            )
            cp.start()
            lcp.append(cp)

        p2 = []
        ocp = []
        for c in range(C):
            p1[c].wait_recv()
            lcp[c].wait()
            sl = (pl.ds(OFFS[c], SIZES[c]), slice(None))
            red_ref[sl] = xh_ref[sl] + comm_ref[sl]
            rows = pl.ds(mine0 + OFFS[c], SIZES[c])
            crows = pl.ds(OFFS[c], SIZES[c])
            rdma = pltpu.make_async_remote_copy(
                src_ref=red_ref.at[crows, :],
                dst_ref=out_ref.at[rows, :],
                send_sem=s2.at[c],
                recv_sem=r2.at[c],
                device_id=y_peer,
                device_id_type=pl.DeviceIdType.MESH,
            )
            rdma.start()
            p2.append(rdma)
            cp = pltpu.make_async_copy(
                red_ref.at[crows, :], out_ref.at[rows, :], lo.at[c]
            )
            cp.start()
            ocp.append(cp)

        for c in range(C):
            rows = pl.ds(other0 + OFFS[c], SIZES[c])
            recv = pltpu.make_async_remote_copy(
                src_ref=red_ref.at[pl.ds(OFFS[c], SIZES[c]), :],
                dst_ref=out_ref.at[rows, :],
                send_sem=s2.at[c],
                recv_sem=r2.at[c],
                device_id=y_peer,
                device_id_type=pl.DeviceIdType.MESH,
            )
            recv.wait_recv()

        for c in range(C):
            ocp[c].wait()
            p1[c].wait_send()
            p2[c].wait_send()

    return pl.pallas_call(
        body,
        out_shape=jax.ShapeDtypeStruct((m, n), x.dtype),
        in_specs=[pl.BlockSpec(memory_space=pl.ANY)],
        out_specs=pl.BlockSpec(memory_space=pl.ANY),
        scratch_shapes=[
            pltpu.VMEM((half, n), x.dtype),
            pltpu.VMEM((half, n), x.dtype),
            pltpu.VMEM((half, n), x.dtype),
            pltpu.SemaphoreType.DMA((C,)),
            pltpu.SemaphoreType.DMA((C,)),
            pltpu.SemaphoreType.DMA((C,)),
            pltpu.SemaphoreType.DMA((C,)),
            pltpu.SemaphoreType.DMA((C,)),
            pltpu.SemaphoreType.DMA((C,)),
        ],
        compiler_params=pltpu.CompilerParams(collective_id=0),
    )(x)
